# baseline (speedup 1.0000x reference)
import functools

import jax
import jax.numpy as jnp
from jax import lax
from jax.experimental import pallas as pl
from jax.experimental.pallas import tpu as pltpu
from jax.experimental.pallas import tpu_sc as plsc

_NULL = 16
_MODAL = 32
_ROW = 64
_NW = 32
_NPAD = 100096
_CHUNK = 128
_NCHUNK = _NPAD // _CHUNK
_KFULL = 48


def _fuse_t_body(vs_ref, vi_ref, ts_ref, ti_ref, out_ref):
    vs = vs_ref[...]
    vi = vi_ref[...]
    ts = ts_ref[...]
    ti = ti_ref[...]
    out_ref[...] = jnp.concatenate(
        [vs[:_NULL], vs[_NULL:] + vi, ts[:_NULL], ts[_NULL:] + ti], axis=0)


def _build_fused_t(v_sem_t, v_id_t, t_sem_t, t_id_t):
    c = 4352
    grid = _NPAD // c
    return pl.pallas_call(
        _fuse_t_body,
        grid=(grid,),
        in_specs=[
            pl.BlockSpec((_MODAL, c), lambda i: (0, i)),
            pl.BlockSpec((_NULL, c), lambda i: (0, i)),
            pl.BlockSpec((_MODAL, c), lambda i: (0, i)),
            pl.BlockSpec((_NULL, c), lambda i: (0, i)),
        ],
        out_specs=pl.BlockSpec((_ROW, c), lambda i: (0, i)),
        out_shape=jax.ShapeDtypeStruct((_ROW, _NPAD), jnp.float32),
    )(v_sem_t, v_id_t, t_sem_t, t_id_t)


def _sc_gather(fused_t4, ids4, hist):
    mesh = plsc.VectorSubcoreMesh(core_axis_name="c", subcore_axis_name="s")
    hg = hist // 8

    @functools.partial(
        pl.kernel,
        mesh=mesh,
        compiler_params=pltpu.CompilerParams(
            use_tc_tiling_on_sc=False, needs_layout_passes=False),
        out_type=(
            jax.ShapeDtypeStruct((hist, 8, _NW, 8, 128), jnp.float32),
            jax.ShapeDtypeStruct((2, _NPAD, _ROW), jnp.float32),
        ),
        scratch_types=(
            [pltpu.VMEM((hg, 8, 128), jnp.int32)]
            + [pltpu.VMEM((8, 1, 8, 128), jnp.float32)] * 2
            + [pltpu.VMEM((_CHUNK, _ROW), jnp.float32)] * 2
            + [pltpu.VMEM((128, _ROW), jnp.float32)] * 2
            + [pltpu.VMEM((8, 8, 128), jnp.float32)] * 2
            + [pltpu.SemaphoreType.DMA] * 9
        ),
    )
    def k(t4_hbm, idx_hbm, out_hbm, scr_hbm, idx_v,
          ib0, ib1, tb0a, tb0b, ra, rb, ta, tb,
          isem, rs0, rs1, ws0, ws1, gs0, gs1, os0, os1):
        ib = (ib0, ib1)
        tb0 = (tb0a, tb0b)
        rows = (ra, rb)
        tbuf = (ta, tb)
        rs = (rs0, rs1)
        ws = (ws0, ws1)
        gs = (gs0, gs1)
        osem = (os0, os1)
        core = lax.axis_index("c")
        sub = lax.axis_index("s")
        wid = sub * 2 + core
        iota = lax.iota(jnp.int32, 16)

        for gg in range(hg):
            pltpu.async_copy(idx_hbm.at[gg, wid], idx_v.at[gg], isem)

        iota_d0 = {d0: iota + d0 for d0 in range(0, _ROW, 16)}
        dg_d0 = {d0: (iota + d0) // 8 for d0 in range(0, _ROW, 16)}
        dl_vec = iota % 8
        zero16 = jnp.zeros((16,), jnp.int32)

        def transpose_chunk(src, dst):
            def bl_body(bl, carry):
                blv = jnp.full((16,), 0, jnp.int32) + bl
                for d0 in range(0, _ROW, 16):
                    vec = plsc.load_gather(
                        src, [dg_d0[d0], zero16, dl_vec, blv])
                    plsc.store_scatter(dst, [blv, iota_d0[d0]], vec)
                return carry
            lax.fori_loop(0, _CHUNK, bl_body, 0)

        def rd(b, j):
            pltpu.async_copy(t4_hbm.at[:, pl.ds(j, 1)], ib[b], rs[b])

        def rd_wait(b):
            pltpu.make_async_copy(t4_hbm.at[:, pl.ds(0, 1)], ib[b], rs[b]).wait()

        def wr(b, j):
            pltpu.async_copy(
                tb0[b], scr_hbm.at[core, pl.ds(_CHUNK * j, _CHUNK)], ws[b])

        def wr_wait(b):
            pltpu.make_async_copy(
                tb0[b], scr_hbm.at[core, pl.ds(0, _CHUNK)], ws[b]).wait()

        rd(0, sub)
        rd(1, sub + 16)

        def p0_body(k0, carry):
            for b in range(2):
                kk = 2 * k0 + b
                j = sub + 16 * kk
                rd_wait(b)
                transpose_chunk(ib[b], tb0[b])
                @pl.when(kk < _KFULL - 2)
                def _():
                    rd(b, sub + 16 * (kk + 2))
                wr(b, j)
                wr_wait(b)
            return carry

        lax.fori_loop(0, _KFULL // 2, p0_body, 0)

        @pl.when(sub < _NCHUNK - 16 * _KFULL)
        def _():
            j = 16 * _KFULL + sub
            pltpu.sync_copy(t4_hbm.at[:, pl.ds(j, 1)], ib[0])
            transpose_chunk(ib[0], tb0[0])
            pltpu.sync_copy(tb0[0], scr_hbm.at[core, pl.ds(_CHUNK * j, _CHUNK)])

        pltpu.make_async_copy(idx_hbm.at[0, 0], idx_v, isem).wait()
        plsc.subcore_barrier()

        def start_g(b, h):
            pltpu.async_copy(
                scr_hbm.at[core].at[idx_v.at[h // 8, h % 8]], rows[b], gs[b])

        def wait_g(b):
            pltpu.make_async_copy(
                scr_hbm.at[core].at[pl.ds(0, 128)], rows[b], gs[b]).wait()

        def start_w(b, h):
            pltpu.async_copy(tbuf[b], out_hbm.at[h, :, wid], osem[b])

        def wait_w(b):
            pltpu.make_async_copy(tbuf[b], out_hbm.at[0, :, wid], osem[b]).wait()

        def transpose_rows(b):
            def bl_body(blg, carry):
                for blu in range(4):
                    blv = jnp.full((16,), 0, jnp.int32) + (blg * 4 + blu)
                    for d0 in range(0, _ROW, 16):
                        vec = plsc.load_gather(rows[b], [blv, iota_d0[d0]])
                        plsc.store_scatter(
                            tbuf[b], [dg_d0[d0], dl_vec, blv], vec)
                return carry
            lax.fori_loop(0, 32, bl_body, 0)

        start_g(0, 0)
        start_g(1, 1)

        def p1_body(h0, carry):
            for b in range(2):
                h = 2 * h0 + b
                wait_g(b)
                transpose_rows(b)
                start_g(b, h + 2)
                start_w(b, h)
                wait_w(b)
            return carry

        lax.fori_loop(0, hist // 2 - 1, p1_body, 0)

        for b in range(2):
            h = hist - 2 + b
            wait_g(b)
            transpose_rows(b)
            start_w(b, h)
        for b in range(2):
            wait_w(b)

    return k(fused_t4, ids4)


def kernel(item_ids, v_sem, v_id, t_sem, t_id):
    batch, hist = item_ids.shape
    fused_t = _build_fused_t(v_sem.T, v_id.T, t_sem.T, t_id.T)
    fused_t4 = fused_t.reshape(8, 8, _NPAD // 128, 128).transpose(0, 2, 1, 3)
    ids4 = (item_ids.T.astype(jnp.int32)
            .reshape(hist // 8, 8, batch // 128, 128).transpose(0, 2, 1, 3))
    out5, _ = _sc_gather(fused_t4, ids4, hist)
    return out5.transpose(2, 4, 0, 1, 3).reshape(batch, hist, _ROW)


# speedup vs baseline: 1.0005x; 1.0005x over previous
import functools

import jax
import jax.numpy as jnp
from jax import lax
from jax.experimental import pallas as pl
from jax.experimental.pallas import tpu as pltpu
from jax.experimental.pallas import tpu_sc as plsc

_NULL = 16
_MODAL = 32
_ROW = 64
_NW = 32
_NPAD = 100096
_CHUNK = 128
_NCHUNK = _NPAD // _CHUNK
_KFULL = 48


def _fuse_t_body(vs_ref, vi_ref, ts_ref, ti_ref, out_ref):
    vs = vs_ref[...]
    vi = vi_ref[...]
    ts = ts_ref[...]
    ti = ti_ref[...]
    out_ref[...] = jnp.concatenate(
        [vs[:_NULL], vs[_NULL:] + vi, ts[:_NULL], ts[_NULL:] + ti], axis=0)


def _build_fused_t(v_sem_t, v_id_t, t_sem_t, t_id_t):
    c = 4352
    grid = _NPAD // c
    return pl.pallas_call(
        _fuse_t_body,
        grid=(grid,),
        in_specs=[
            pl.BlockSpec((_MODAL, c), lambda i: (0, i)),
            pl.BlockSpec((_NULL, c), lambda i: (0, i)),
            pl.BlockSpec((_MODAL, c), lambda i: (0, i)),
            pl.BlockSpec((_NULL, c), lambda i: (0, i)),
        ],
        out_specs=pl.BlockSpec((_ROW, c), lambda i: (0, i)),
        out_shape=jax.ShapeDtypeStruct((_ROW, _NPAD), jnp.float32),
    )(v_sem_t, v_id_t, t_sem_t, t_id_t)


def _sc_gather(fused_t4, ids4, hist):
    mesh = plsc.VectorSubcoreMesh(core_axis_name="c", subcore_axis_name="s")
    hg = hist // 8

    @functools.partial(
        pl.kernel,
        mesh=mesh,
        compiler_params=pltpu.CompilerParams(
            use_tc_tiling_on_sc=False, needs_layout_passes=False),
        out_type=(
            jax.ShapeDtypeStruct((hist, 8, _NW, 8, 128), jnp.float32),
            jax.ShapeDtypeStruct((2, _NPAD, _ROW), jnp.float32),
        ),
        scratch_types=(
            [pltpu.VMEM((hg, 8, 128), jnp.int32)]
            + [pltpu.VMEM((8, 1, 8, 128), jnp.float32)] * 2
            + [pltpu.VMEM((_CHUNK, _ROW), jnp.float32)] * 2
            + [pltpu.VMEM((128, _ROW), jnp.float32)] * 2
            + [pltpu.VMEM((8, 8, 128), jnp.float32)] * 2
            + [pltpu.SemaphoreType.DMA] * 9
        ),
    )
    def k(t4_hbm, idx_hbm, out_hbm, scr_hbm, idx_v,
          ib0, ib1, tb0a, tb0b, ra, rb, ta, tb,
          isem, rs0, rs1, ws0, ws1, gs0, gs1, os0, os1):
        ib = (ib0, ib1)
        tb0 = (tb0a, tb0b)
        rows = (ra, rb)
        tbuf = (ta, tb)
        rs = (rs0, rs1)
        ws = (ws0, ws1)
        gs = (gs0, gs1)
        osem = (os0, os1)
        core = lax.axis_index("c")
        sub = lax.axis_index("s")
        wid = sub * 2 + core
        iota = lax.iota(jnp.int32, 16)

        for gg in range(hg):
            pltpu.async_copy(idx_hbm.at[gg, wid], idx_v.at[gg], isem)

        iota_d0 = {d0: iota + d0 for d0 in range(0, _ROW, 16)}
        dg_d0 = {d0: (iota + d0) // 8 for d0 in range(0, _ROW, 16)}
        dl_vec = iota % 8
        zero16 = jnp.zeros((16,), jnp.int32)

        def transpose_chunk(src, dst):
            def bl_body(bl, carry):
                blv = jnp.full((16,), 0, jnp.int32) + bl
                for d0 in range(0, _ROW, 16):
                    vec = plsc.load_gather(
                        src, [dg_d0[d0], zero16, dl_vec, blv])
                    plsc.store_scatter(dst, [blv, iota_d0[d0]], vec)
                return carry
            lax.fori_loop(0, _CHUNK, bl_body, 0)

        def rd(b, j):
            pltpu.async_copy(t4_hbm.at[:, pl.ds(j, 1)], ib[b], rs[b])

        def rd_wait(b):
            pltpu.make_async_copy(t4_hbm.at[:, pl.ds(0, 1)], ib[b], rs[b]).wait()

        def wr(b, j):
            pltpu.async_copy(
                tb0[b], scr_hbm.at[core, pl.ds(_CHUNK * j, _CHUNK)], ws[b])

        def wr_wait(b):
            pltpu.make_async_copy(
                tb0[b], scr_hbm.at[core, pl.ds(0, _CHUNK)], ws[b]).wait()

        rd(0, sub)
        rd(1, sub + 16)

        scope = jax.named_scope

        def p0_body(k0, carry):
            for b in range(2):
                kk = 2 * k0 + b
                j = sub + 16 * kk
                rd_wait(b)
                transpose_chunk(ib[b], tb0[b])
                @pl.when(kk < _KFULL - 2)
                def _():
                    rd(b, sub + 16 * (kk + 2))
                wr(b, j)
                wr_wait(b)
            return carry

        with scope("phase0_main"):
            lax.fori_loop(0, _KFULL // 2, p0_body, 0)

        @pl.when(sub < _NCHUNK - 16 * _KFULL)
        def _():
            j = 16 * _KFULL + sub
            pltpu.sync_copy(t4_hbm.at[:, pl.ds(j, 1)], ib[0])
            transpose_chunk(ib[0], tb0[0])
            pltpu.sync_copy(tb0[0], scr_hbm.at[core, pl.ds(_CHUNK * j, _CHUNK)])

        with scope("idx_wait_barrier"):
            pltpu.make_async_copy(idx_hbm.at[0, 0], idx_v, isem).wait()
            plsc.subcore_barrier()

        def start_g(b, h):
            pltpu.async_copy(
                scr_hbm.at[core].at[idx_v.at[h // 8, h % 8]], rows[b], gs[b])

        def wait_g(b):
            pltpu.make_async_copy(
                scr_hbm.at[core].at[pl.ds(0, 128)], rows[b], gs[b]).wait()

        def start_w(b, h):
            pltpu.async_copy(tbuf[b], out_hbm.at[h, :, wid], osem[b])

        def wait_w(b):
            pltpu.make_async_copy(tbuf[b], out_hbm.at[0, :, wid], osem[b]).wait()

        def transpose_rows(b):
            def bl_body(blg, carry):
                for blu in range(4):
                    blv = jnp.full((16,), 0, jnp.int32) + (blg * 4 + blu)
                    for d0 in range(0, _ROW, 16):
                        vec = plsc.load_gather(rows[b], [blv, iota_d0[d0]])
                        plsc.store_scatter(
                            tbuf[b], [dg_d0[d0], dl_vec, blv], vec)
                return carry
            lax.fori_loop(0, 32, bl_body, 0)

        start_g(0, 0)
        start_g(1, 1)

        def p1_body(h0, carry):
            for b in range(2):
                h = 2 * h0 + b
                wait_g(b)
                transpose_rows(b)
                start_g(b, h + 2)
                start_w(b, h)
                wait_w(b)
            return carry

        with scope("phase1_main"):
            lax.fori_loop(0, hist // 2 - 1, p1_body, 0)

        for b in range(2):
            h = hist - 2 + b
            wait_g(b)
            transpose_rows(b)
            start_w(b, h)
        for b in range(2):
            wait_w(b)

    return k(fused_t4, ids4)


def kernel(item_ids, v_sem, v_id, t_sem, t_id):
    batch, hist = item_ids.shape
    fused_t = _build_fused_t(v_sem.T, v_id.T, t_sem.T, t_id.T)
    fused_t4 = fused_t.reshape(8, 8, _NPAD // 128, 128).transpose(0, 2, 1, 3)
    ids4 = (item_ids.T.astype(jnp.int32)
            .reshape(hist // 8, 8, batch // 128, 128).transpose(0, 2, 1, 3))
    out5, _ = _sc_gather(fused_t4, ids4, hist)
    return out5.transpose(2, 4, 0, 1, 3).reshape(batch, hist, _ROW)


# speedup vs baseline: 1.1828x; 1.1823x over previous
import functools

import jax
import jax.numpy as jnp
from jax import lax
from jax.experimental import pallas as pl
from jax.experimental.pallas import tpu as pltpu
from jax.experimental.pallas import tpu_sc as plsc

_NULL = 16
_MODAL = 32
_ROW = 64
_NW = 32
_NPAD = 100096
_CHUNK = 128
_NCHUNK = _NPAD // _CHUNK
_KFULL = 48


def _fuse_t_body(vs_ref, vi_ref, ts_ref, ti_ref, out_ref):
    vs = vs_ref[...]
    vi = vi_ref[...]
    ts = ts_ref[...]
    ti = ti_ref[...]
    out_ref[...] = jnp.concatenate(
        [vs[:_NULL], vs[_NULL:] + vi, ts[:_NULL], ts[_NULL:] + ti], axis=0)


def _build_fused_t(v_sem_t, v_id_t, t_sem_t, t_id_t):
    c = 4352
    grid = _NPAD // c
    return pl.pallas_call(
        _fuse_t_body,
        grid=(grid,),
        in_specs=[
            pl.BlockSpec((_MODAL, c), lambda i: (0, i)),
            pl.BlockSpec((_NULL, c), lambda i: (0, i)),
            pl.BlockSpec((_MODAL, c), lambda i: (0, i)),
            pl.BlockSpec((_NULL, c), lambda i: (0, i)),
        ],
        out_specs=pl.BlockSpec((_ROW, c), lambda i: (0, i)),
        out_shape=jax.ShapeDtypeStruct((_ROW, _NPAD), jnp.float32),
    )(v_sem_t, v_id_t, t_sem_t, t_id_t)


def _sc_gather(fused_t4, ids4, hist):
    mesh = plsc.VectorSubcoreMesh(core_axis_name="c", subcore_axis_name="s")
    hg = hist // 8

    @functools.partial(
        pl.kernel,
        mesh=mesh,
        compiler_params=pltpu.CompilerParams(
            use_tc_tiling_on_sc=False, needs_layout_passes=False),
        out_type=(
            jax.ShapeDtypeStruct((hist, 8, _NW, 8, 128), jnp.float32),
            jax.ShapeDtypeStruct((2, _NPAD, _ROW), jnp.float32),
        ),
        scratch_types=(
            [pltpu.VMEM((hg, 8, 128), jnp.int32)]
            + [pltpu.VMEM((8, 1, 8, 128), jnp.float32)] * 2
            + [pltpu.VMEM((_CHUNK, _ROW), jnp.float32)] * 2
            + [pltpu.VMEM((128, _ROW), jnp.float32)] * 2
            + [pltpu.VMEM((8, 8, 128), jnp.float32)] * 4
            + [pltpu.SemaphoreType.DMA] * 11
        ),
    )
    def k(t4_hbm, idx_hbm, out_hbm, scr_hbm, idx_v,
          ib0, ib1, tb0a, tb0b, ra, rb, ta, tb, tc, td,
          isem, rs0, rs1, ws0, ws1, gs0, gs1, os0, os1, os2, os3):
        ib = (ib0, ib1)
        tb0 = (tb0a, tb0b)
        rows = (ra, rb)
        tbuf = (ta, tb, tc, td)
        rs = (rs0, rs1)
        ws = (ws0, ws1)
        gs = (gs0, gs1)
        osem = (os0, os1, os2, os3)
        core = lax.axis_index("c")
        sub = lax.axis_index("s")
        wid = sub * 2 + core
        iota = lax.iota(jnp.int32, 16)

        for gg in range(hg):
            pltpu.async_copy(idx_hbm.at[gg, wid], idx_v.at[gg], isem)

        iota_d0 = {d0: iota + d0 for d0 in range(0, _ROW, 16)}
        dg_d0 = {d0: (iota + d0) // 8 for d0 in range(0, _ROW, 16)}
        dl_vec = iota % 8
        zero16 = jnp.zeros((16,), jnp.int32)

        def transpose_chunk(src, dst):
            def bl_body(blg, carry):
                blvs = [zero16 + (blg * 4 + blu) for blu in range(4)]
                vecs = [
                    plsc.load_gather(src, [dg_d0[d0], zero16, dl_vec, blv])
                    for blv in blvs for d0 in range(0, _ROW, 16)
                ]
                i = 0
                for blv in blvs:
                    for d0 in range(0, _ROW, 16):
                        plsc.store_scatter(dst, [blv, iota_d0[d0]], vecs[i])
                        i += 1
                return carry
            lax.fori_loop(0, _CHUNK // 4, bl_body, 0)

        def rd(b, j):
            pltpu.async_copy(t4_hbm.at[:, pl.ds(j, 1)], ib[b], rs[b])

        def rd_wait(b):
            pltpu.make_async_copy(t4_hbm.at[:, pl.ds(0, 1)], ib[b], rs[b]).wait()

        def wr(b, j):
            pltpu.async_copy(
                tb0[b], scr_hbm.at[core, pl.ds(_CHUNK * j, _CHUNK)], ws[b])

        def wr_wait(b):
            pltpu.make_async_copy(
                tb0[b], scr_hbm.at[core, pl.ds(0, _CHUNK)], ws[b]).wait()

        rd(0, sub)
        rd(1, sub + 16)

        scope = jax.named_scope

        with scope("phase0_main"):
            for kk in range(2):
                rd_wait(kk)
                transpose_chunk(ib[kk], tb0[kk])
                rd(kk, sub + 16 * (kk + 2))
                wr(kk, sub + 16 * kk)

            def p0_body(k0, carry):
                for b in range(2):
                    kk = 2 + 2 * k0 + b
                    j = sub + 16 * kk
                    rd_wait(b)
                    wr_wait(b)
                    transpose_chunk(ib[b], tb0[b])
                    @pl.when(kk < _KFULL - 2)
                    def _():
                        rd(b, sub + 16 * (kk + 2))
                    wr(b, j)
                return carry

            lax.fori_loop(0, _KFULL // 2 - 1, p0_body, 0)
            wr_wait(0)
            wr_wait(1)

        @pl.when(sub < _NCHUNK - 16 * _KFULL)
        def _():
            j = 16 * _KFULL + sub
            pltpu.sync_copy(t4_hbm.at[:, pl.ds(j, 1)], ib[0])
            transpose_chunk(ib[0], tb0[0])
            pltpu.sync_copy(tb0[0], scr_hbm.at[core, pl.ds(_CHUNK * j, _CHUNK)])

        with scope("idx_wait_barrier"):
            pltpu.make_async_copy(idx_hbm.at[0, 0], idx_v, isem).wait()
            plsc.subcore_barrier()

        def start_g(b, h):
            pltpu.async_copy(
                scr_hbm.at[core].at[idx_v.at[h // 8, h % 8]], rows[b], gs[b])

        def wait_g(b):
            pltpu.make_async_copy(
                scr_hbm.at[core].at[pl.ds(0, 128)], rows[b], gs[b]).wait()

        def start_w(b, h):
            pltpu.async_copy(tbuf[b], out_hbm.at[h, :, wid], osem[b])

        def wait_w(b):
            pltpu.make_async_copy(tbuf[b], out_hbm.at[0, :, wid], osem[b]).wait()

        def transpose_rows(bg, bt):
            def bl_body(blg, carry):
                blvs = [zero16 + (blg * 4 + blu) for blu in range(4)]
                vecs = [
                    plsc.load_gather(rows[bg], [blv, iota_d0[d0]])
                    for blv in blvs for d0 in range(0, _ROW, 16)
                ]
                i = 0
                for blv in blvs:
                    for d0 in range(0, _ROW, 16):
                        plsc.store_scatter(
                            tbuf[bt], [dg_d0[d0], dl_vec, blv], vecs[i])
                        i += 1
                return carry
            lax.fori_loop(0, 32, bl_body, 0)

        start_g(0, 0)
        start_g(1, 1)

        with scope("phase1_main"):
            for h in range(4):
                wait_g(h % 2)
                transpose_rows(h % 2, h)
                start_g(h % 2, h + 2)
                start_w(h, h)

            def p1_body(k, carry):
                for i in range(4):
                    h = 4 + 4 * k + i
                    wait_g(i % 2)
                    wait_w(i)
                    transpose_rows(i % 2, i)
                    start_g(i % 2, h + 2)
                    start_w(i, h)
                return carry

            lax.fori_loop(0, (hist - 8) // 4, p1_body, 0)

        for h in range(hist - 4, hist):
            bg, bt = h % 2, h % 4
            wait_g(bg)
            wait_w(bt)
            transpose_rows(bg, bt)
            if h + 2 < hist:
                start_g(bg, h + 2)
            start_w(bt, h)
        for bt in range(4):
            wait_w(bt)

    return k(fused_t4, ids4)


def kernel(item_ids, v_sem, v_id, t_sem, t_id):
    batch, hist = item_ids.shape
    fused_t = _build_fused_t(v_sem.T, v_id.T, t_sem.T, t_id.T)
    fused_t4 = fused_t.reshape(8, 8, _NPAD // 128, 128).transpose(0, 2, 1, 3)
    ids4 = (item_ids.T.astype(jnp.int32)
            .reshape(hist // 8, 8, batch // 128, 128).transpose(0, 2, 1, 3))
    out5, _ = _sc_gather(fused_t4, ids4, hist)
    return out5.transpose(2, 4, 0, 1, 3).reshape(batch, hist, _ROW)


# speedup vs baseline: 3.8999x; 3.2972x over previous
import functools

import jax
import jax.numpy as jnp
from jax import lax
from jax.experimental import pallas as pl
from jax.experimental.pallas import tpu as pltpu
from jax.experimental.pallas import tpu_sc as plsc

_NULL = 16
_MODAL = 32
_ROW = 64
_NW = 32
_NPAD = 100096
_CHUNK = 128
_NCHUNK = _NPAD // _CHUNK
_KFULL = 48


def _fuse_t_body(vs_ref, vi_ref, ts_ref, ti_ref, out_ref):
    vs = vs_ref[...]
    vi = vi_ref[...]
    ts = ts_ref[...]
    ti = ti_ref[...]
    out_ref[...] = jnp.concatenate(
        [vs[:_NULL], vs[_NULL:] + vi, ts[:_NULL], ts[_NULL:] + ti], axis=0)


def _build_fused_t(v_sem_t, v_id_t, t_sem_t, t_id_t):
    c = 4352
    grid = _NPAD // c
    return pl.pallas_call(
        _fuse_t_body,
        grid=(grid,),
        in_specs=[
            pl.BlockSpec((_MODAL, c), lambda i: (0, i)),
            pl.BlockSpec((_NULL, c), lambda i: (0, i)),
            pl.BlockSpec((_MODAL, c), lambda i: (0, i)),
            pl.BlockSpec((_NULL, c), lambda i: (0, i)),
        ],
        out_specs=pl.BlockSpec((_ROW, c), lambda i: (0, i)),
        out_shape=jax.ShapeDtypeStruct((_ROW, _NPAD), jnp.float32),
    )(v_sem_t, v_id_t, t_sem_t, t_id_t)


def _sc_gather(fused_t4, ids4, hist):
    mesh = plsc.VectorSubcoreMesh(core_axis_name="c", subcore_axis_name="s")
    hg = hist // 8

    @functools.partial(
        pl.kernel,
        mesh=mesh,
        compiler_params=pltpu.CompilerParams(
            use_tc_tiling_on_sc=False, needs_layout_passes=False),
        out_type=(
            jax.ShapeDtypeStruct((hist, 8, _NW, 8, 128), jnp.float32),
            jax.ShapeDtypeStruct((2, _NPAD, _ROW), jnp.float32),
        ),
        scratch_types=(
            [pltpu.VMEM((hg, 8, 128), jnp.int32)]
            + [pltpu.VMEM((8, 1, 8, 129), jnp.float32)] * 2
            + [pltpu.VMEM((_CHUNK, _ROW), jnp.float32)] * 2
            + [pltpu.VMEM((128, _ROW), jnp.float32)] * 2
            + [pltpu.VMEM((8, 8, 129), jnp.float32)] * 4
            + [pltpu.SemaphoreType.DMA] * 11
        ),
    )
    def k(t4_hbm, idx_hbm, out_hbm, scr_hbm, idx_v,
          ib0, ib1, tb0a, tb0b, ra, rb, ta, tb, tc, td,
          isem, rs0, rs1, ws0, ws1, gs0, gs1, os0, os1, os2, os3):
        ib = (ib0, ib1)
        tb0 = (tb0a, tb0b)
        rows = (ra, rb)
        tbuf = (ta, tb, tc, td)
        rs = (rs0, rs1)
        ws = (ws0, ws1)
        gs = (gs0, gs1)
        osem = (os0, os1, os2, os3)
        core = lax.axis_index("c")
        sub = lax.axis_index("s")
        wid = sub * 2 + core
        iota = lax.iota(jnp.int32, 16)

        for gg in range(hg):
            pltpu.async_copy(idx_hbm.at[gg, wid], idx_v.at[gg], isem)

        iota_d0 = {d0: iota + d0 for d0 in range(0, _ROW, 16)}
        dg_d0 = {d0: (iota + d0) // 8 for d0 in range(0, _ROW, 16)}
        dl_vec = iota % 8
        zero16 = jnp.zeros((16,), jnp.int32)

        def transpose_chunk(src, dst):
            def bl_body(blg, carry):
                blvs = [zero16 + (blg * 4 + blu) for blu in range(4)]
                vecs = [
                    plsc.load_gather(src, [dg_d0[d0], zero16, dl_vec, blv])
                    for blv in blvs for d0 in range(0, _ROW, 16)
                ]
                i = 0
                for blv in blvs:
                    for d0 in range(0, _ROW, 16):
                        plsc.store_scatter(dst, [blv, iota_d0[d0]], vecs[i])
                        i += 1
                return carry
            lax.fori_loop(0, _CHUNK // 4, bl_body, 0)

        def rd(b, j):
            pltpu.async_copy(t4_hbm.at[:, pl.ds(j, 1)],
                             ib[b].at[:, :, :, pl.ds(0, 128)], rs[b])

        def rd_wait(b):
            pltpu.make_async_copy(t4_hbm.at[:, pl.ds(0, 1)],
                                  ib[b].at[:, :, :, pl.ds(0, 128)], rs[b]).wait()

        def wr(b, j):
            pltpu.async_copy(
                tb0[b], scr_hbm.at[core, pl.ds(_CHUNK * j, _CHUNK)], ws[b])

        def wr_wait(b):
            pltpu.make_async_copy(
                tb0[b], scr_hbm.at[core, pl.ds(0, _CHUNK)], ws[b]).wait()

        rd(0, sub)
        rd(1, sub + 16)

        scope = jax.named_scope

        with scope("phase0_main"):
            for kk in range(2):
                rd_wait(kk)
                transpose_chunk(ib[kk], tb0[kk])
                rd(kk, sub + 16 * (kk + 2))
                wr(kk, sub + 16 * kk)

            def p0_body(k0, carry):
                for b in range(2):
                    kk = 2 + 2 * k0 + b
                    j = sub + 16 * kk
                    rd_wait(b)
                    wr_wait(b)
                    transpose_chunk(ib[b], tb0[b])
                    @pl.when(kk < _KFULL - 2)
                    def _():
                        rd(b, sub + 16 * (kk + 2))
                    wr(b, j)
                return carry

            lax.fori_loop(0, _KFULL // 2 - 1, p0_body, 0)
            wr_wait(0)
            wr_wait(1)

        @pl.when(sub < _NCHUNK - 16 * _KFULL)
        def _():
            j = 16 * _KFULL + sub
            pltpu.sync_copy(t4_hbm.at[:, pl.ds(j, 1)],
                            ib[0].at[:, :, :, pl.ds(0, 128)])
            transpose_chunk(ib[0], tb0[0])
            pltpu.sync_copy(tb0[0], scr_hbm.at[core, pl.ds(_CHUNK * j, _CHUNK)])

        with scope("idx_wait_barrier"):
            pltpu.make_async_copy(idx_hbm.at[0, 0], idx_v, isem).wait()
            plsc.subcore_barrier()

        def start_g(b, h):
            pltpu.async_copy(
                scr_hbm.at[core].at[idx_v.at[h // 8, h % 8]], rows[b], gs[b])

        def wait_g(b):
            pltpu.make_async_copy(
                scr_hbm.at[core].at[pl.ds(0, 128)], rows[b], gs[b]).wait()

        def start_w(b, h):
            pltpu.async_copy(tbuf[b].at[:, :, pl.ds(0, 128)],
                             out_hbm.at[h, :, wid], osem[b])

        def wait_w(b):
            pltpu.make_async_copy(tbuf[b].at[:, :, pl.ds(0, 128)],
                                  out_hbm.at[0, :, wid], osem[b]).wait()

        def transpose_rows(bg, bt):
            def bl_body(blg, carry):
                blvs = [zero16 + (blg * 4 + blu) for blu in range(4)]
                vecs = [
                    plsc.load_gather(rows[bg], [blv, iota_d0[d0]])
                    for blv in blvs for d0 in range(0, _ROW, 16)
                ]
                i = 0
                for blv in blvs:
                    for d0 in range(0, _ROW, 16):
                        plsc.store_scatter(
                            tbuf[bt], [dg_d0[d0], dl_vec, blv], vecs[i])
                        i += 1
                return carry
            lax.fori_loop(0, 32, bl_body, 0)

        start_g(0, 0)
        start_g(1, 1)

        with scope("phase1_main"):
            for h in range(4):
                wait_g(h % 2)
                transpose_rows(h % 2, h)
                start_g(h % 2, h + 2)
                start_w(h, h)

            def p1_body(k, carry):
                for i in range(4):
                    h = 4 + 4 * k + i
                    wait_g(i % 2)
                    wait_w(i)
                    transpose_rows(i % 2, i)
                    start_g(i % 2, h + 2)
                    start_w(i, h)
                return carry

            lax.fori_loop(0, (hist - 8) // 4, p1_body, 0)

        for h in range(hist - 4, hist):
            bg, bt = h % 2, h % 4
            wait_g(bg)
            wait_w(bt)
            transpose_rows(bg, bt)
            if h + 2 < hist:
                start_g(bg, h + 2)
            start_w(bt, h)
        for bt in range(4):
            wait_w(bt)

    return k(fused_t4, ids4)


def kernel(item_ids, v_sem, v_id, t_sem, t_id):
    batch, hist = item_ids.shape
    fused_t = _build_fused_t(v_sem.T, v_id.T, t_sem.T, t_id.T)
    fused_t4 = fused_t.reshape(8, 8, _NPAD // 128, 128).transpose(0, 2, 1, 3)
    ids4 = (item_ids.T.astype(jnp.int32)
            .reshape(hist // 8, 8, batch // 128, 128).transpose(0, 2, 1, 3))
    out5, _ = _sc_gather(fused_t4, ids4, hist)
    return out5.transpose(2, 4, 0, 1, 3).reshape(batch, hist, _ROW)
